# Initial kernel scaffold; baseline (speedup 1.0000x reference)
import jax
import jax.numpy as jnp
from jax.experimental import pallas as pl


def kernel(x, edge_index, edge_weight, W1, W2):
    raise NotImplementedError("write your pallas kernel here")


# speedup vs baseline: 1.3726x; 1.3726x over previous
import functools

import jax
import jax.numpy as jnp
from jax import lax
from jax.experimental import pallas as pl
from jax.experimental.pallas import tpu as pltpu
from jax.experimental.pallas import tpu_sc as plsc

_N = 10000
_E = 160000
_NC = 2
_NS = 16
_NPC = _N // _NC
_RPT = 320
_DUMMY = _RPT * _NS
_ACC_ROWS = _DUMMY + 8
_K = 80
_NBLK = 128
_EPT = _NBLK * _K
_EPAD = _EPT * _NS
_LAST = _NPC - _RPT * (_NS - 1)


def _mp_body(Dg, G, *refs):
    hw_parts = refs[:G]
    (src_v2, dst_v2, ew_1d, out,
     src_v, idx_v, ew_v, rows0, rows1, acc, sem0, sem1) = refs[G:]
    c = lax.axis_index("c")
    s = lax.axis_index("s")
    row0 = s * _RPT
    blk0 = s * _NBLK

    pltpu.sync_copy(src_v2.at[pl.ds(blk0, _NBLK)], src_v)
    pltpu.sync_copy(dst_v2.at[pl.ds(blk0, _NBLK)], idx_v)
    pltpu.sync_copy(ew_1d.at[pl.ds(s * _EPT, _EPT)], ew_v.at[pl.ds(0, _EPT)])

    lo = c * _NPC

    def _mk_idx(j, carry):
        for t in range(_K // 16):
            v = idx_v[j, pl.ds(t * 16, 16)]
            loc = v - lo
            ok = (loc >= 0) & (loc < _NPC)
            idx_v[j, pl.ds(t * 16, 16)] = jnp.where(ok, loc, _DUMMY)
        return carry

    lax.fori_loop(0, _NBLK, _mk_idx, 0)

    zero = jnp.zeros((16,), jnp.float32)

    for g in range(G):
        hw = hw_parts[g]

        def _zero_buf(e, carry):
            for d in range(Dg // 16):
                rows0[e, pl.ds(d * 16, 16)] = zero
            return carry

        lax.fori_loop(0, _K, _zero_buf, 0)
        for q in range(_RPT // _K):
            pltpu.sync_copy(rows0, acc.at[pl.ds(row0 + q * _K, _K)])

        plsc.subcore_barrier()

        def _issue(j, buf, sem):
            pltpu.async_copy(hw.at[src_v.at[j]], buf, sem)

        def _wait(buf, sem):
            pltpu.make_async_copy(hw.at[src_v.at[0]], buf, sem).wait()

        def _process(j, buf):
            def _scale(e, carry):
                ew16 = jnp.full((16,), ew_v[pl.ds(j * _K + e, 16)][0],
                                jnp.float32)
                for d in range(Dg // 16):
                    buf[e, pl.ds(d * 16, 16)] = (
                        buf[e, pl.ds(d * 16, 16)] * ew16)
                return carry

            lax.fori_loop(0, _K, _scale, 0)
            pltpu.sync_copy(buf, acc.at[idx_v.at[j]], add=True)

        _issue(0, rows0, sem0)

        def _blk(j, carry):
            nxt = j + 1

            @pl.when(j % 2 == 0)
            def _():
                _wait(rows0, sem0)

                @pl.when(nxt < _NBLK)
                def _():
                    _issue(nxt, rows1, sem1)

                _process(j, rows0)

            @pl.when(j % 2 == 1)
            def _():
                _wait(rows1, sem1)

                @pl.when(nxt < _NBLK)
                def _():
                    _issue(nxt, rows0, sem0)

                _process(j, rows1)

            return carry

        lax.fori_loop(0, _NBLK, _blk, 0)

        plsc.subcore_barrier()

        @pl.when(s < _NS - 1)
        def _():
            pltpu.sync_copy(acc.at[pl.ds(row0, _RPT)],
                            out.at[pl.ds(lo + row0, _RPT),
                                   pl.ds(g * Dg, Dg)])

        @pl.when(s == _NS - 1)
        def _():
            pltpu.sync_copy(acc.at[pl.ds((_NS - 1) * _RPT, _LAST)],
                            out.at[pl.ds(lo + (_NS - 1) * _RPT, _LAST),
                                   pl.ds(g * Dg, Dg)])


@functools.lru_cache(maxsize=None)
def _make_mp(Dg, G):
    mesh = plsc.VectorSubcoreMesh(core_axis_name="c", subcore_axis_name="s")
    return functools.partial(
        pl.kernel,
        mesh=mesh,
        out_type=jax.ShapeDtypeStruct((_N, Dg * G), jnp.float32),
        scratch_types=[
            pltpu.VMEM((_NBLK, _K), jnp.int32),
            pltpu.VMEM((_NBLK, _K), jnp.int32),
            pltpu.VMEM((_EPT + 16,), jnp.float32),
            pltpu.VMEM((_K, Dg), jnp.float32),
            pltpu.VMEM((_K, Dg), jnp.float32),
            pltpu.VMEM_SHARED((_ACC_ROWS, Dg), jnp.float32),
            pltpu.SemaphoreType.DMA,
            pltpu.SemaphoreType.DMA,
        ],
    )(functools.partial(_mp_body, Dg, G))


def _mm_block(relu, a_ref, b_ref, o_ref):
    a = a_ref[...]
    if relu:
        a = jnp.maximum(a, 0.0)
    o_ref[...] = jnp.dot(a, b_ref[...], preferred_element_type=jnp.float32)


def _mm(a, b, relu=False, bn=1000):
    n, k = a.shape
    h = b.shape[1]
    return pl.pallas_call(
        functools.partial(_mm_block, relu),
        grid=(n // bn,),
        in_specs=[pl.BlockSpec((bn, k), lambda i: (i, 0)),
                  pl.BlockSpec((k, h), lambda i: (0, 0))],
        out_specs=pl.BlockSpec((bn, h), lambda i: (i, 0)),
        out_shape=jax.ShapeDtypeStruct((n, h), jnp.float32),
    )(a, b)


def _gram_block(zi_ref, zj_ref, o_ref):
    o_ref[...] = lax.dot_general(
        zi_ref[...], zj_ref[...], (((1,), (1,)), ((), ())),
        preferred_element_type=jnp.float32)


def _gram(z, bz=200):
    n, h = z.shape
    return pl.pallas_call(
        _gram_block,
        grid=(n // bz,),
        in_specs=[pl.BlockSpec((bz, h), lambda i: (i, 0)),
                  pl.BlockSpec((n, h), lambda i: (0, 0))],
        out_specs=pl.BlockSpec((bz, n), lambda i: (i, 0)),
        out_shape=jax.ShapeDtypeStruct((n, n), jnp.float32),
    )(z, z)


def kernel(x, edge_index, edge_weight, W1, W2):
    pad = _EPAD - _E
    src2 = jnp.pad(edge_index[0], (0, pad)).reshape(_EPAD // _K, _K)
    dst2 = jnp.pad(edge_index[1], (0, pad)).reshape(_EPAD // _K, _K)
    ew1 = jnp.pad(edge_weight, (0, pad))

    hw1 = _mm(x, W1)
    agg1 = _make_mp(128, 2)(hw1[:, :128], hw1[:, 128:], src2, dst2, ew1)
    hw2 = jnp.pad(_mm(agg1, W2, relu=True), ((0, 0), (0, 64)))
    z = _make_mp(128, 1)(hw2, src2, dst2, ew1)[:, :64]
    return _gram(z).reshape(-1)


# speedup vs baseline: 1.4118x; 1.0286x over previous
import functools

import jax
import jax.numpy as jnp
from jax import lax
from jax.experimental import pallas as pl
from jax.experimental.pallas import tpu as pltpu
from jax.experimental.pallas import tpu_sc as plsc

_N = 10000
_E = 160000
_NC = 2
_NS = 16
_NPC = _N // _NC
_RPT = 320
_DUMMY = _RPT * _NS
_ACC_ROWS = _DUMMY + 8
_K = 128
_NBLK = 80
_EPT = _NBLK * _K
_EPAD = _EPT * _NS
_LAST = _NPC - _RPT * (_NS - 1)


def _mp_body(Dg, G, *refs):
    hw_parts = refs[:G]
    (src_v2, dst_v2, ew_1d, out,
     src_v, idx_v, ew_v, rows0, rows1, acc,
     sem0, sem1, sem2, sem3) = refs[G:]
    c = lax.axis_index("c")
    s = lax.axis_index("s")
    row0 = s * _RPT
    blk0 = s * _NBLK

    pltpu.sync_copy(src_v2.at[pl.ds(blk0, _NBLK)], src_v)
    pltpu.sync_copy(dst_v2.at[pl.ds(blk0, _NBLK)], idx_v)
    pltpu.sync_copy(ew_1d.at[pl.ds(s * _EPT, _EPT)], ew_v.at[pl.ds(0, _EPT)])

    lo = c * _NPC

    def _mk_idx(j, carry):
        for t in range(_K // 16):
            v = idx_v[j, pl.ds(t * 16, 16)]
            loc = v - lo
            ok = (loc >= 0) & (loc < _NPC)
            idx_v[j, pl.ds(t * 16, 16)] = jnp.where(ok, loc, _DUMMY)
        return carry

    lax.fori_loop(0, _NBLK, _mk_idx, 0)

    zero = jnp.zeros((16,), jnp.float32)

    for g in range(G):
        hw = hw_parts[g]

        def _zero_buf(e, carry):
            for d in range(Dg // 16):
                rows0[e, pl.ds(d * 16, 16)] = zero
            return carry

        lax.fori_loop(0, _K, _zero_buf, 0)
        pltpu.sync_copy(rows0, acc.at[pl.ds(row0, _K)])
        pltpu.sync_copy(rows0, acc.at[pl.ds(row0 + _K, _K)])
        pltpu.sync_copy(rows0.at[pl.ds(0, _RPT - 2 * _K)],
                        acc.at[pl.ds(row0 + 2 * _K, _RPT - 2 * _K)])

        plsc.subcore_barrier()

        def _issue(j, buf, gsem):
            pltpu.async_copy(hw.at[src_v.at[j]], buf, gsem)

        def _gwait(buf, gsem):
            pltpu.make_async_copy(hw.at[src_v.at[0]], buf, gsem).wait()

        def _scat(j, buf, ssem):
            pltpu.async_copy(buf, acc.at[idx_v.at[j]], ssem, add=True)

        def _swait(buf, ssem):
            pltpu.make_async_copy(buf, acc.at[idx_v.at[0]], ssem).wait()

        def _scale(j, buf):
            def _grp(q, carry):
                ews = ew_v[pl.ds(j * _K + q * 16, 16)]
                for i in range(16):
                    e = q * 16 + i
                    ew16 = lax.gather(
                        ews, jnp.full((16, 1), i, jnp.int32),
                        lax.GatherDimensionNumbers(
                            offset_dims=(), collapsed_slice_dims=(0,),
                            start_index_map=(0,)),
                        slice_sizes=(1,),
                        mode=lax.GatherScatterMode.PROMISE_IN_BOUNDS)
                    for d in range(Dg // 16):
                        buf[e, pl.ds(d * 16, 16)] = (
                            buf[e, pl.ds(d * 16, 16)] * ew16)
                return carry

            lax.fori_loop(0, _K // 16, _grp, 0)

        _issue(0, rows0, sem0)

        def _half(j, buf, obuf, gsem, ogsem, ssem, ossem):
            nxt = j + 1
            _gwait(buf, gsem)

            @pl.when(j >= 1)
            def _():
                _swait(obuf, ossem)

            @pl.when(nxt < _NBLK)
            def _():
                _issue(nxt, obuf, ogsem)

            _scale(j, buf)
            _scat(j, buf, ssem)

        def _blk(j, carry):
            @pl.when(j % 2 == 0)
            def _():
                _half(j, rows0, rows1, sem0, sem1, sem2, sem3)

            @pl.when(j % 2 == 1)
            def _():
                _half(j, rows1, rows0, sem1, sem0, sem3, sem2)

            return carry

        lax.fori_loop(0, _NBLK, _blk, 0)
        _swait(rows1, sem3)

        plsc.subcore_barrier()

        @pl.when(s < _NS - 1)
        def _():
            pltpu.sync_copy(acc.at[pl.ds(row0, _RPT)],
                            out.at[pl.ds(lo + row0, _RPT),
                                   pl.ds(g * Dg, Dg)])

        @pl.when(s == _NS - 1)
        def _():
            pltpu.sync_copy(acc.at[pl.ds((_NS - 1) * _RPT, _LAST)],
                            out.at[pl.ds(lo + (_NS - 1) * _RPT, _LAST),
                                   pl.ds(g * Dg, Dg)])


@functools.lru_cache(maxsize=None)
def _make_mp(Dg, G):
    mesh = plsc.VectorSubcoreMesh(core_axis_name="c", subcore_axis_name="s")
    return functools.partial(
        pl.kernel,
        mesh=mesh,
        out_type=jax.ShapeDtypeStruct((_N, Dg * G), jnp.float32),
        scratch_types=[
            pltpu.VMEM((_NBLK, _K), jnp.int32),
            pltpu.VMEM((_NBLK, _K), jnp.int32),
            pltpu.VMEM((_EPT + 16,), jnp.float32),
            pltpu.VMEM((_K, Dg), jnp.float32),
            pltpu.VMEM((_K, Dg), jnp.float32),
            pltpu.VMEM_SHARED((_ACC_ROWS, Dg), jnp.float32),
            pltpu.SemaphoreType.DMA,
            pltpu.SemaphoreType.DMA,
            pltpu.SemaphoreType.DMA,
            pltpu.SemaphoreType.DMA,
        ],
    )(functools.partial(_mp_body, Dg, G))


def _mm_block(relu, a_ref, b_ref, o_ref):
    a = a_ref[...]
    if relu:
        a = jnp.maximum(a, 0.0)
    o_ref[...] = jnp.dot(a, b_ref[...], preferred_element_type=jnp.float32)


def _mm(a, b, relu=False, bn=1000):
    n, k = a.shape
    h = b.shape[1]
    return pl.pallas_call(
        functools.partial(_mm_block, relu),
        grid=(n // bn,),
        in_specs=[pl.BlockSpec((bn, k), lambda i: (i, 0)),
                  pl.BlockSpec((k, h), lambda i: (0, 0))],
        out_specs=pl.BlockSpec((bn, h), lambda i: (i, 0)),
        out_shape=jax.ShapeDtypeStruct((n, h), jnp.float32),
    )(a, b)


def _gram_block(zi_ref, zj_ref, o_ref):
    o_ref[...] = lax.dot_general(
        zi_ref[...], zj_ref[...], (((1,), (1,)), ((), ())),
        preferred_element_type=jnp.float32)


def _gram(z, bz=200):
    n, h = z.shape
    return pl.pallas_call(
        _gram_block,
        grid=(n // bz,),
        in_specs=[pl.BlockSpec((bz, h), lambda i: (i, 0)),
                  pl.BlockSpec((n, h), lambda i: (0, 0))],
        out_specs=pl.BlockSpec((bz, n), lambda i: (i, 0)),
        out_shape=jax.ShapeDtypeStruct((n, n), jnp.float32),
    )(z, z)


def kernel(x, edge_index, edge_weight, W1, W2):
    pad = _EPAD - _E
    src2 = jnp.pad(edge_index[0], (0, pad)).reshape(_EPAD // _K, _K)
    dst2 = jnp.pad(edge_index[1], (0, pad)).reshape(_EPAD // _K, _K)
    ew1 = jnp.pad(edge_weight, (0, pad))

    hw1 = _mm(x, W1)
    agg1 = _make_mp(128, 2)(hw1[:, :128], hw1[:, 128:], src2, dst2, ew1)
    hw2 = jnp.pad(_mm(agg1, W2, relu=True), ((0, 0), (0, 64)))
    z = _make_mp(128, 1)(hw2, src2, dst2, ew1)[:, :64]
    return _gram(z).reshape(-1)
